# baseline (speedup 1.0000x reference)
import functools

import jax
import jax.numpy as jnp
from jax import lax
from jax.experimental import pallas as pl
from jax.experimental.pallas import tpu as pltpu
from jax.experimental.pallas import tpu_sc as plsc

N = 10000
E = 160000
D_NODE = 256
D_EDGE = 16
D_GLOB = 64
D_MID = 512
D_OUT = 256
N_GRAPHS = 8

NSLAB = 4
FSL = D_MID // NSLAB
NC = 2
NT = 16
NPAD = 10240
ROWS_PER_TILE = NPAD // NT
EDGES_PER_TILE = E // (NC * NT)
CH = 64
NCHUNK = 80
E_PAD = NC * NT * NCHUNK * CH
EDGES_PER_TILE_PAD = NCHUNK * CH
DUMP_ROW = NPAD - 1

BN = 1000
BE = 1024


def _xw1_body(x_ref, w1a_ref, b1_ref, o0, o1, o2, o3):
    xb = x_ref[...]
    outs = (o0, o1, o2, o3)
    for s in range(NSLAB):
        w = w1a_ref[:, s * FSL:(s + 1) * FSL]
        outs[s][...] = (
            jnp.dot(xb, w, preferred_element_type=jnp.float32)
            + b1_ref[:, s * FSL:(s + 1) * FSL]
        )


def _ew_body(ea_ref, w1b_ref, o_ref):
    eb = ea_ref[...]
    for s in range(NSLAB):
        w = w1b_ref[:, s * FSL:(s + 1) * FSL]
        o_ref[s, :, :] = jnp.dot(eb, w, preferred_element_type=jnp.float32)


def _sc_body(xw0, xw1s, xw2, xw3, ew, row3, col3, ids, ones_h, zeros_h,
             spart, cnt,
             rowv, colv, idbuf, gbuf, ebuf, slab, sem):
    cidx = lax.axis_index("c")
    sidx = lax.axis_index("s")
    t0 = sidx * ROWS_PER_TILE
    tbase = (cidx * NT + sidx) * EDGES_PER_TILE_PAD
    xw = (xw0, xw1s, xw2, xw3)

    pltpu.sync_copy(row3.at[cidx, sidx], rowv)
    pltpu.sync_copy(col3.at[cidx, sidx], colv)

    def zero_slab():
        pltpu.sync_copy(zeros_h, ebuf)
        for j in range(ROWS_PER_TILE // CH):
            pltpu.sync_copy(ids.at[pl.ds(t0 + j * CH, CH)], idbuf)
            pltpu.sync_copy(ebuf, slab.at[idbuf])
        plsc.subcore_barrier()

    def readout(dst):
        plsc.subcore_barrier()
        for j in range(ROWS_PER_TILE // CH):
            pltpu.sync_copy(ids.at[pl.ds(t0 + j * CH, CH)], idbuf)
            pltpu.async_copy(slab.at[idbuf], gbuf, sem).wait()
            pltpu.sync_copy(gbuf, dst.at[pl.ds(t0 + j * CH, CH)])

    for p in range(NSLAB):
        zero_slab()

        def gbody(g, carry):
            pltpu.sync_copy(ew.at[p, pl.ds(tbase + g * CH, CH)], ebuf)
            pltpu.async_copy(xw[p].at[rowv.at[g]], gbuf, sem).wait()

            def rbody(r, carry2):
                for k in range(FSL // 16):
                    sl = pl.ds(k * 16, 16)
                    gbuf[r, sl] = jnp.maximum(gbuf[r, sl] + ebuf[r, sl], 0.0)
                return carry2

            lax.fori_loop(0, CH, rbody, 0)
            pltpu.sync_copy(gbuf, slab.at[colv.at[g]], add=True)
            return carry

        lax.fori_loop(0, NCHUNK, gbody, 0)
        readout(spart.at[cidx, p])

    zero_slab()
    pltpu.sync_copy(ones_h, ebuf)

    def cbody(g, carry):
        pltpu.sync_copy(ebuf, slab.at[colv.at[g]], add=True)
        return carry

    lax.fori_loop(0, NCHUNK, cbody, 0)
    readout(cnt.at[cidx])


@functools.cache
def _sc_scatter_kernel():
    return pl.kernel(
        _sc_body,
        out_type=[
            jax.ShapeDtypeStruct((NC, NSLAB, NPAD, FSL), jnp.float32),
            jax.ShapeDtypeStruct((NC, NPAD, FSL), jnp.float32),
        ],
        mesh=plsc.VectorSubcoreMesh(core_axis_name="c", subcore_axis_name="s",
                                    num_cores=NC, num_subcores=NT),
        scratch_types=[
            pltpu.VMEM((NCHUNK, CH), jnp.int32),
            pltpu.VMEM((NCHUNK, CH), jnp.int32),
            pltpu.VMEM((CH,), jnp.int32),
            pltpu.VMEM((CH, FSL), jnp.float32),
            pltpu.VMEM((CH, FSL), jnp.float32),
            pltpu.VMEM_SHARED((NPAD, FSL), jnp.float32),
            pltpu.SemaphoreType.DMA,
        ],
    )


def _post_body(sp_ref, cnt_ref, x_ref, bf_ref, u_ref, w2_ref, b2_ref,
               w3_ref, b3_ref, w4_ref, b4_ref, o_ref):
    acc = jnp.zeros((BN, D_MID), jnp.float32)
    for c in range(NC):
        for p in range(NSLAB):
            acc += jnp.dot(sp_ref[c, p], w2_ref[p * FSL:(p + 1) * FSL, :],
                           preferred_element_type=jnp.float32)
    cntv = cnt_ref[0] + cnt_ref[1]
    c1 = cntv[:, 0:1]
    summed = acc + c1 * b2_ref[...]
    mean = summed / jnp.maximum(c1, 1.0)

    ub = jnp.dot(u_ref[...], w3_ref[D_NODE + D_MID:, :],
                 preferred_element_type=jnp.float32)
    b = bf_ref[0]
    iota = lax.broadcasted_iota(jnp.int32, (N_GRAPHS, BN), 0).astype(jnp.float32)
    ohT = (iota == b).astype(jnp.float32)
    g = lax.dot_general(ohT, ub, (((0,), (0,)), ((), ())),
                        preferred_element_type=jnp.float32)

    h = jnp.dot(x_ref[...], w3_ref[:D_NODE, :],
                preferred_element_type=jnp.float32)
    h += jnp.dot(mean, w3_ref[D_NODE:D_NODE + D_MID, :],
                 preferred_element_type=jnp.float32)
    h = jnp.maximum(h + g + b3_ref[...], 0.0)
    o_ref[...] = jnp.dot(h, w4_ref[...],
                         preferred_element_type=jnp.float32) + b4_ref[...]


def kernel(x, edge_index, edge_attr, u, batch, W1, b1, W2, b2, W3, b3, W4, b4):
    row = edge_index[0].astype(jnp.int32)
    col = edge_index[1].astype(jnp.int32)
    npad_e = E_PAD - E
    row3 = jnp.concatenate([row, jnp.zeros((npad_e,), jnp.int32)])
    row3 = row3.reshape(NC, NT, NCHUNK, CH)
    col3 = jnp.concatenate([col, jnp.full((npad_e,), DUMP_ROW, jnp.int32)])
    col3 = col3.reshape(NC, NT, NCHUNK, CH)
    ea_pad = jnp.concatenate(
        [edge_attr, jnp.zeros((npad_e, D_EDGE), jnp.float32)])
    W1a = W1[:D_NODE]
    W1b = W1[D_NODE:]
    b1r = b1.reshape(1, D_MID)

    xw_slabs = pl.pallas_call(
        _xw1_body,
        grid=(N // BN,),
        in_specs=[
            pl.BlockSpec((BN, D_NODE), lambda i: (i, 0)),
            pl.BlockSpec((D_NODE, D_MID), lambda i: (0, 0)),
            pl.BlockSpec((1, D_MID), lambda i: (0, 0)),
        ],
        out_specs=[pl.BlockSpec((BN, FSL), lambda i: (i, 0))] * NSLAB,
        out_shape=[jax.ShapeDtypeStruct((N, FSL), jnp.float32)] * NSLAB,
    )(x, W1a, b1r)

    ew = pl.pallas_call(
        _ew_body,
        grid=(E_PAD // BE,),
        in_specs=[
            pl.BlockSpec((BE, D_EDGE), lambda i: (i, 0)),
            pl.BlockSpec((D_EDGE, D_MID), lambda i: (0, 0)),
        ],
        out_specs=pl.BlockSpec((NSLAB, BE, FSL), lambda i: (0, i, 0)),
        out_shape=jax.ShapeDtypeStruct((NSLAB, E_PAD, FSL), jnp.float32),
    )(ea_pad, W1b)

    ids = jnp.arange(NPAD, dtype=jnp.int32)
    ones_h = jnp.ones((CH, FSL), jnp.float32)
    zeros_h = jnp.zeros((CH, FSL), jnp.float32)

    spart, cnt = _sc_scatter_kernel()(xw_slabs[0], xw_slabs[1], xw_slabs[2],
                                      xw_slabs[3], ew, row3, col3, ids,
                                      ones_h, zeros_h)

    batchf = batch.astype(jnp.float32).reshape(N // BN, 1, BN)
    b2r = b2.reshape(1, D_MID)
    b3r = b3.reshape(1, D_MID)
    b4r = b4.reshape(1, D_OUT)

    out = pl.pallas_call(
        _post_body,
        grid=(N // BN,),
        in_specs=[
            pl.BlockSpec((NC, NSLAB, BN, FSL), lambda i: (0, 0, i, 0)),
            pl.BlockSpec((NC, BN, FSL), lambda i: (0, i, 0)),
            pl.BlockSpec((BN, D_NODE), lambda i: (i, 0)),
            pl.BlockSpec((1, 1, BN), lambda i: (i, 0, 0)),
            pl.BlockSpec((N_GRAPHS, D_GLOB), lambda i: (0, 0)),
            pl.BlockSpec((D_MID, D_MID), lambda i: (0, 0)),
            pl.BlockSpec((1, D_MID), lambda i: (0, 0)),
            pl.BlockSpec((D_NODE + D_MID + D_GLOB, D_MID), lambda i: (0, 0)),
            pl.BlockSpec((1, D_MID), lambda i: (0, 0)),
            pl.BlockSpec((D_MID, D_OUT), lambda i: (0, 0)),
            pl.BlockSpec((1, D_OUT), lambda i: (0, 0)),
        ],
        out_specs=pl.BlockSpec((BN, D_OUT), lambda i: (i, 0)),
        out_shape=jax.ShapeDtypeStruct((N, D_OUT), jnp.float32),
    )(spart, cnt, x, batchf, u, W2, b2r, W3, b3r, W4, b4r)

    return out


# speedup vs baseline: 1.3887x; 1.3887x over previous
import functools

import jax
import jax.numpy as jnp
from jax import lax
from jax.experimental import pallas as pl
from jax.experimental.pallas import tpu as pltpu
from jax.experimental.pallas import tpu_sc as plsc

N = 10000
E = 160000
D_NODE = 256
D_EDGE = 16
D_GLOB = 64
D_MID = 512
D_OUT = 256
N_GRAPHS = 8

NSLAB = 4
FSL = D_MID // NSLAB
NC = 2
NT = 16
NPAD = 10240
ROWS_PER_TILE = NPAD // NT
EDGES_PER_TILE = E // (NC * NT)
CH = 64
NFULL = EDGES_PER_TILE // CH
TAIL = EDGES_PER_TILE - NFULL * CH

BN = 1000
BE = 1000


def _xw1_body(x_ref, w1a_ref, b1_ref, o0, o1, o2, o3):
    xb = x_ref[...]
    outs = (o0, o1, o2, o3)
    for s in range(NSLAB):
        w = w1a_ref[:, s * FSL:(s + 1) * FSL]
        outs[s][...] = (
            jnp.dot(xb, w, preferred_element_type=jnp.float32)
            + b1_ref[:, s * FSL:(s + 1) * FSL]
        )


def _ew_body(ea_ref, w1b_ref, o_ref):
    eb = ea_ref[...]
    for s in range(NSLAB):
        w = w1b_ref[:, s * FSL:(s + 1) * FSL]
        o_ref[s, :, :] = jnp.dot(eb, w, preferred_element_type=jnp.float32)


def _sc_body(xw0, xw1s, xw2, xw3, ew, row, col, ids, ones_h, zeros_h,
             spart, cnt,
             idx_r, idx_c, idx_rt, idx_ct, idbuf, gbuf, ebuf, gbuft, ebuft,
             slab, sem):
    cidx = lax.axis_index("c")
    sidx = lax.axis_index("s")
    t0 = sidx * ROWS_PER_TILE
    ebase = cidx * (E // NC) + sidx * EDGES_PER_TILE
    xw = (xw0, xw1s, xw2, xw3)

    def zero_slab():
        pltpu.sync_copy(zeros_h, ebuf)
        for j in range(ROWS_PER_TILE // CH):
            pltpu.sync_copy(ids.at[pl.ds(t0 + j * CH, CH)], idbuf)
            pltpu.sync_copy(ebuf, slab.at[idbuf])
        plsc.subcore_barrier()

    def readout(dst):
        plsc.subcore_barrier()
        for j in range(ROWS_PER_TILE // CH):
            pltpu.sync_copy(ids.at[pl.ds(t0 + j * CH, CH)], idbuf)
            pltpu.async_copy(slab.at[idbuf], gbuf, sem).wait()
            pltpu.sync_copy(gbuf, dst.at[pl.ds(t0 + j * CH, CH)])

    def process(e0, ir, ic, gb, eb, ch, p):
        pltpu.sync_copy(row.at[pl.ds(e0, ch)], ir)
        pltpu.sync_copy(col.at[pl.ds(e0, ch)], ic)
        gather = pltpu.async_copy(xw[p].at[ir], gb, sem)
        pltpu.sync_copy(ew.at[p, pl.ds(e0, ch)], eb)
        gather.wait()

        def rbody(r, carry):
            for k in range(FSL // 16):
                sl = pl.ds(k * 16, 16)
                gb[r, sl] = jnp.maximum(gb[r, sl] + eb[r, sl], 0.0)
            return carry

        lax.fori_loop(0, ch, rbody, 0)
        pltpu.sync_copy(gb, slab.at[ic], add=True)

    for p in range(NSLAB):
        zero_slab()

        def gbody(g, carry):
            process(ebase + g * CH, idx_r, idx_c, gbuf, ebuf, CH, p)
            return carry

        lax.fori_loop(0, NFULL, gbody, 0)
        process(ebase + NFULL * CH, idx_rt, idx_ct, gbuft, ebuft, TAIL, p)
        readout(spart.at[cidx, p])

    zero_slab()
    pltpu.sync_copy(ones_h, ebuf)
    pltpu.sync_copy(ones_h.at[pl.ds(0, TAIL)], ebuft)

    def cbody(g, carry):
        e0 = ebase + g * CH
        pltpu.sync_copy(col.at[pl.ds(e0, CH)], idx_c)
        pltpu.sync_copy(ebuf, slab.at[idx_c], add=True)
        return carry

    lax.fori_loop(0, NFULL, cbody, 0)
    pltpu.sync_copy(col.at[pl.ds(ebase + NFULL * CH, TAIL)], idx_ct)
    pltpu.sync_copy(ebuft, slab.at[idx_ct], add=True)
    readout(cnt.at[cidx])


@functools.cache
def _sc_scatter_kernel():
    return pl.kernel(
        _sc_body,
        out_type=[
            jax.ShapeDtypeStruct((NC, NSLAB, NPAD, FSL), jnp.float32),
            jax.ShapeDtypeStruct((NC, NPAD, FSL), jnp.float32),
        ],
        mesh=plsc.VectorSubcoreMesh(core_axis_name="c", subcore_axis_name="s",
                                    num_cores=NC, num_subcores=NT),
        scratch_types=[
            pltpu.VMEM((CH,), jnp.int32),
            pltpu.VMEM((CH,), jnp.int32),
            pltpu.VMEM((TAIL,), jnp.int32),
            pltpu.VMEM((TAIL,), jnp.int32),
            pltpu.VMEM((CH,), jnp.int32),
            pltpu.VMEM((CH, FSL), jnp.float32),
            pltpu.VMEM((CH, FSL), jnp.float32),
            pltpu.VMEM((TAIL, FSL), jnp.float32),
            pltpu.VMEM((TAIL, FSL), jnp.float32),
            pltpu.VMEM_SHARED((NPAD, FSL), jnp.float32),
            pltpu.SemaphoreType.DMA,
        ],
    )


def _post_body(sp_ref, cnt_ref, x_ref, bf_ref, u_ref, w2_ref, b2_ref,
               w3_ref, b3_ref, w4_ref, b4_ref, o_ref):
    acc = jnp.zeros((BN, D_MID), jnp.float32)
    for c in range(NC):
        for p in range(NSLAB):
            acc += jnp.dot(sp_ref[c, p], w2_ref[p * FSL:(p + 1) * FSL, :],
                           preferred_element_type=jnp.float32)
    cntv = cnt_ref[0] + cnt_ref[1]
    c1 = cntv[:, 0:1]
    summed = acc + c1 * b2_ref[...]
    mean = summed / jnp.maximum(c1, 1.0)

    ub = jnp.dot(u_ref[...], w3_ref[D_NODE + D_MID:, :],
                 preferred_element_type=jnp.float32)
    b = bf_ref[0]
    iota = lax.broadcasted_iota(jnp.int32, (N_GRAPHS, BN), 0).astype(jnp.float32)
    ohT = (iota == b).astype(jnp.float32)
    g = lax.dot_general(ohT, ub, (((0,), (0,)), ((), ())),
                        preferred_element_type=jnp.float32)

    h = jnp.dot(x_ref[...], w3_ref[:D_NODE, :],
                preferred_element_type=jnp.float32)
    h += jnp.dot(mean, w3_ref[D_NODE:D_NODE + D_MID, :],
                 preferred_element_type=jnp.float32)
    h = jnp.maximum(h + g + b3_ref[...], 0.0)
    o_ref[...] = jnp.dot(h, w4_ref[...],
                         preferred_element_type=jnp.float32) + b4_ref[...]


def kernel(x, edge_index, edge_attr, u, batch, W1, b1, W2, b2, W3, b3, W4, b4):
    row = edge_index[0].astype(jnp.int32)
    col = edge_index[1].astype(jnp.int32)
    W1a = W1[:D_NODE]
    W1b = W1[D_NODE:]
    b1r = b1.reshape(1, D_MID)

    xw_slabs = pl.pallas_call(
        _xw1_body,
        grid=(N // BN,),
        in_specs=[
            pl.BlockSpec((BN, D_NODE), lambda i: (i, 0)),
            pl.BlockSpec((D_NODE, D_MID), lambda i: (0, 0)),
            pl.BlockSpec((1, D_MID), lambda i: (0, 0)),
        ],
        out_specs=[pl.BlockSpec((BN, FSL), lambda i: (i, 0))] * NSLAB,
        out_shape=[jax.ShapeDtypeStruct((N, FSL), jnp.float32)] * NSLAB,
    )(x, W1a, b1r)

    ew = pl.pallas_call(
        _ew_body,
        grid=(E // BE,),
        in_specs=[
            pl.BlockSpec((BE, D_EDGE), lambda i: (i, 0)),
            pl.BlockSpec((D_EDGE, D_MID), lambda i: (0, 0)),
        ],
        out_specs=pl.BlockSpec((NSLAB, BE, FSL), lambda i: (0, i, 0)),
        out_shape=jax.ShapeDtypeStruct((NSLAB, E, FSL), jnp.float32),
    )(edge_attr, W1b)

    ids = jnp.arange(NPAD, dtype=jnp.int32)
    ones_h = jnp.ones((CH, FSL), jnp.float32)
    zeros_h = jnp.zeros((CH, FSL), jnp.float32)

    spart, cnt = _sc_scatter_kernel()(xw_slabs[0], xw_slabs[1], xw_slabs[2],
                                      xw_slabs[3], ew, row, col, ids,
                                      ones_h, zeros_h)

    batchf = batch.astype(jnp.float32).reshape(N // BN, 1, BN)
    b2r = b2.reshape(1, D_MID)
    b3r = b3.reshape(1, D_MID)
    b4r = b4.reshape(1, D_OUT)

    out = pl.pallas_call(
        _post_body,
        grid=(N // BN,),
        in_specs=[
            pl.BlockSpec((NC, NSLAB, BN, FSL), lambda i: (0, 0, i, 0)),
            pl.BlockSpec((NC, BN, FSL), lambda i: (0, i, 0)),
            pl.BlockSpec((BN, D_NODE), lambda i: (i, 0)),
            pl.BlockSpec((1, 1, BN), lambda i: (i, 0, 0)),
            pl.BlockSpec((N_GRAPHS, D_GLOB), lambda i: (0, 0)),
            pl.BlockSpec((D_MID, D_MID), lambda i: (0, 0)),
            pl.BlockSpec((1, D_MID), lambda i: (0, 0)),
            pl.BlockSpec((D_NODE + D_MID + D_GLOB, D_MID), lambda i: (0, 0)),
            pl.BlockSpec((1, D_MID), lambda i: (0, 0)),
            pl.BlockSpec((D_MID, D_OUT), lambda i: (0, 0)),
            pl.BlockSpec((1, D_OUT), lambda i: (0, 0)),
        ],
        out_specs=pl.BlockSpec((BN, D_OUT), lambda i: (i, 0)),
        out_shape=jax.ShapeDtypeStruct((N, D_OUT), jnp.float32),
    )(spart, cnt, x, batchf, u, W2, b2r, W3, b3r, W4, b4r)

    return out


# speedup vs baseline: 1.6623x; 1.1971x over previous
import functools

import jax
import jax.numpy as jnp
from jax import lax
from jax.experimental import pallas as pl
from jax.experimental.pallas import tpu as pltpu
from jax.experimental.pallas import tpu_sc as plsc

N = 10000
E = 160000
D_NODE = 256
D_EDGE = 16
D_GLOB = 64
D_MID = 512
D_OUT = 256
N_GRAPHS = 8

NSLAB = 4
FSL = D_MID // NSLAB
NC = 2
NT = 16
NPAD = 10240
ROWS_PER_TILE = NPAD // NT
EDGES_PER_TILE = E // (NC * NT)
CH = 128
NFULL = EDGES_PER_TILE // CH
TAIL = EDGES_PER_TILE - NFULL * CH

BN = 1000
BE = 1000


def _xw1_body(x_ref, w1a_ref, b1_ref, o0, o1, o2, o3):
    xb = x_ref[...]
    outs = (o0, o1, o2, o3)
    for s in range(NSLAB):
        w = w1a_ref[:, s * FSL:(s + 1) * FSL]
        outs[s][...] = (
            jnp.dot(xb, w, preferred_element_type=jnp.float32)
            + b1_ref[:, s * FSL:(s + 1) * FSL]
        )


def _ew_body(ea_ref, w1b_ref, o_ref):
    eb = ea_ref[...]
    for s in range(NSLAB):
        w = w1b_ref[:, s * FSL:(s + 1) * FSL]
        o_ref[s, :, :] = jnp.dot(eb, w, preferred_element_type=jnp.float32)


def _sc_body(xw0, xw1s, xw2, xw3, ew, row, col, ids, ones_h, zeros_h,
             spart, cnt,
             idx_r, idx_c, idx_rt, idx_ct, idbuf, gbuf, ebuf, gbuft, ebuft,
             slab, sem):
    cidx = lax.axis_index("c")
    sidx = lax.axis_index("s")
    t0 = sidx * ROWS_PER_TILE
    ebase = cidx * (E // NC) + sidx * EDGES_PER_TILE
    xw = (xw0, xw1s, xw2, xw3)

    def zero_slab():
        pltpu.sync_copy(zeros_h, ebuf)
        for j in range(ROWS_PER_TILE // CH):
            pltpu.sync_copy(ids.at[pl.ds(t0 + j * CH, CH)], idbuf)
            pltpu.sync_copy(ebuf, slab.at[idbuf])
        plsc.subcore_barrier()

    def readout(dst):
        plsc.subcore_barrier()
        for j in range(ROWS_PER_TILE // CH):
            pltpu.sync_copy(ids.at[pl.ds(t0 + j * CH, CH)], idbuf)
            pltpu.async_copy(slab.at[idbuf], gbuf, sem).wait()
            pltpu.sync_copy(gbuf, dst.at[pl.ds(t0 + j * CH, CH)])

    def process(e0, ir, ic, gb, eb, ch, p):
        pltpu.sync_copy(row.at[pl.ds(e0, ch)], ir)
        pltpu.sync_copy(col.at[pl.ds(e0, ch)], ic)
        gather = pltpu.async_copy(xw[p].at[ir], gb, sem)
        pltpu.sync_copy(ew.at[p, pl.ds(e0, ch)], eb)
        gather.wait()

        def rbody(r, carry):
            for k in range(FSL // 16):
                sl = pl.ds(k * 16, 16)
                gb[r, sl] = jnp.maximum(gb[r, sl] + eb[r, sl], 0.0)
            return carry

        lax.fori_loop(0, ch, rbody, 0)
        pltpu.sync_copy(gb, slab.at[ic], add=True)

    for p in range(NSLAB):
        zero_slab()

        def gbody(g, carry):
            process(ebase + g * CH, idx_r, idx_c, gbuf, ebuf, CH, p)
            return carry

        lax.fori_loop(0, NFULL, gbody, 0)
        process(ebase + NFULL * CH, idx_rt, idx_ct, gbuft, ebuft, TAIL, p)
        readout(spart.at[cidx, p])

    zero_slab()
    pltpu.sync_copy(ones_h, ebuf)
    pltpu.sync_copy(ones_h.at[pl.ds(0, TAIL)], ebuft)

    def cbody(g, carry):
        e0 = ebase + g * CH
        pltpu.sync_copy(col.at[pl.ds(e0, CH)], idx_c)
        pltpu.sync_copy(ebuf, slab.at[idx_c], add=True)
        return carry

    lax.fori_loop(0, NFULL, cbody, 0)
    pltpu.sync_copy(col.at[pl.ds(ebase + NFULL * CH, TAIL)], idx_ct)
    pltpu.sync_copy(ebuft, slab.at[idx_ct], add=True)
    readout(cnt.at[cidx])


@functools.cache
def _sc_scatter_kernel():
    return pl.kernel(
        _sc_body,
        out_type=[
            jax.ShapeDtypeStruct((NC, NSLAB, NPAD, FSL), jnp.float32),
            jax.ShapeDtypeStruct((NC, NPAD, FSL), jnp.float32),
        ],
        mesh=plsc.VectorSubcoreMesh(core_axis_name="c", subcore_axis_name="s",
                                    num_cores=NC, num_subcores=NT),
        scratch_types=[
            pltpu.VMEM((CH,), jnp.int32),
            pltpu.VMEM((CH,), jnp.int32),
            pltpu.VMEM((TAIL,), jnp.int32),
            pltpu.VMEM((TAIL,), jnp.int32),
            pltpu.VMEM((CH,), jnp.int32),
            pltpu.VMEM((CH, FSL), jnp.float32),
            pltpu.VMEM((CH, FSL), jnp.float32),
            pltpu.VMEM((TAIL, FSL), jnp.float32),
            pltpu.VMEM((TAIL, FSL), jnp.float32),
            pltpu.VMEM_SHARED((NPAD, FSL), jnp.float32),
            pltpu.SemaphoreType.DMA,
        ],
    )


def _post_body(sp_ref, cnt_ref, x_ref, bf_ref, u_ref, w2_ref, b2_ref,
               w3_ref, b3_ref, w4_ref, b4_ref, o_ref):
    acc = jnp.zeros((BN, D_MID), jnp.float32)
    for c in range(NC):
        for p in range(NSLAB):
            acc += jnp.dot(sp_ref[c, p], w2_ref[p * FSL:(p + 1) * FSL, :],
                           preferred_element_type=jnp.float32)
    cntv = cnt_ref[0] + cnt_ref[1]
    c1 = cntv[:, 0:1]
    summed = acc + c1 * b2_ref[...]
    mean = summed / jnp.maximum(c1, 1.0)

    ub = jnp.dot(u_ref[...], w3_ref[D_NODE + D_MID:, :],
                 preferred_element_type=jnp.float32)
    b = bf_ref[0]
    iota = lax.broadcasted_iota(jnp.int32, (N_GRAPHS, BN), 0).astype(jnp.float32)
    ohT = (iota == b).astype(jnp.float32)
    g = lax.dot_general(ohT, ub, (((0,), (0,)), ((), ())),
                        preferred_element_type=jnp.float32)

    h = jnp.dot(x_ref[...], w3_ref[:D_NODE, :],
                preferred_element_type=jnp.float32)
    h += jnp.dot(mean, w3_ref[D_NODE:D_NODE + D_MID, :],
                 preferred_element_type=jnp.float32)
    h = jnp.maximum(h + g + b3_ref[...], 0.0)
    o_ref[...] = jnp.dot(h, w4_ref[...],
                         preferred_element_type=jnp.float32) + b4_ref[...]


def kernel(x, edge_index, edge_attr, u, batch, W1, b1, W2, b2, W3, b3, W4, b4):
    row = edge_index[0].astype(jnp.int32)
    col = edge_index[1].astype(jnp.int32)
    W1a = W1[:D_NODE]
    W1b = W1[D_NODE:]
    b1r = b1.reshape(1, D_MID)

    xw_slabs = pl.pallas_call(
        _xw1_body,
        grid=(N // BN,),
        in_specs=[
            pl.BlockSpec((BN, D_NODE), lambda i: (i, 0)),
            pl.BlockSpec((D_NODE, D_MID), lambda i: (0, 0)),
            pl.BlockSpec((1, D_MID), lambda i: (0, 0)),
        ],
        out_specs=[pl.BlockSpec((BN, FSL), lambda i: (i, 0))] * NSLAB,
        out_shape=[jax.ShapeDtypeStruct((N, FSL), jnp.float32)] * NSLAB,
    )(x, W1a, b1r)

    ew = pl.pallas_call(
        _ew_body,
        grid=(E // BE,),
        in_specs=[
            pl.BlockSpec((BE, D_EDGE), lambda i: (i, 0)),
            pl.BlockSpec((D_EDGE, D_MID), lambda i: (0, 0)),
        ],
        out_specs=pl.BlockSpec((NSLAB, BE, FSL), lambda i: (0, i, 0)),
        out_shape=jax.ShapeDtypeStruct((NSLAB, E, FSL), jnp.float32),
    )(edge_attr, W1b)

    ids = jnp.arange(NPAD, dtype=jnp.int32)
    ones_h = jnp.ones((CH, FSL), jnp.float32)
    zeros_h = jnp.zeros((CH, FSL), jnp.float32)

    spart, cnt = _sc_scatter_kernel()(xw_slabs[0], xw_slabs[1], xw_slabs[2],
                                      xw_slabs[3], ew, row, col, ids,
                                      ones_h, zeros_h)

    batchf = batch.astype(jnp.float32).reshape(N // BN, 1, BN)
    b2r = b2.reshape(1, D_MID)
    b3r = b3.reshape(1, D_MID)
    b4r = b4.reshape(1, D_OUT)

    out = pl.pallas_call(
        _post_body,
        grid=(N // BN,),
        in_specs=[
            pl.BlockSpec((NC, NSLAB, BN, FSL), lambda i: (0, 0, i, 0)),
            pl.BlockSpec((NC, BN, FSL), lambda i: (0, i, 0)),
            pl.BlockSpec((BN, D_NODE), lambda i: (i, 0)),
            pl.BlockSpec((1, 1, BN), lambda i: (i, 0, 0)),
            pl.BlockSpec((N_GRAPHS, D_GLOB), lambda i: (0, 0)),
            pl.BlockSpec((D_MID, D_MID), lambda i: (0, 0)),
            pl.BlockSpec((1, D_MID), lambda i: (0, 0)),
            pl.BlockSpec((D_NODE + D_MID + D_GLOB, D_MID), lambda i: (0, 0)),
            pl.BlockSpec((1, D_MID), lambda i: (0, 0)),
            pl.BlockSpec((D_MID, D_OUT), lambda i: (0, 0)),
            pl.BlockSpec((1, D_OUT), lambda i: (0, 0)),
        ],
        out_specs=pl.BlockSpec((BN, D_OUT), lambda i: (i, 0)),
        out_shape=jax.ShapeDtypeStruct((N, D_OUT), jnp.float32),
    )(spart, cnt, x, batchf, u, W2, b2r, W3, b3r, W4, b4r)

    return out
